# baseline (speedup 1.0000x reference)
import functools

import jax
import jax.numpy as jnp
from jax import lax
from jax.experimental import pallas as pl
from jax.experimental.pallas import tpu as pltpu
from jax.experimental.pallas import tpu_sc as plsc

_B = 4096
_L = 20
_D = 16
_C = 3 * _D
_BL = _B * _L
_NW = 32
_B_W = _B // _NW
_PASS_B = 32
_NPASS = _B_W // _PASS_B
_RP = _PASS_B * _L
_GROUPS = _RP // 16


@functools.partial(
    pl.kernel,
    mesh=plsc.VectorSubcoreMesh(core_axis_name="c", subcore_axis_name="s"),
    out_type=jax.ShapeDtypeStruct((_L * _C, _B), jnp.float32),
    compiler_params=pltpu.CompilerParams(
        use_tc_tiling_on_sc=False, needs_layout_passes=False),
    scratch_types=[
        pltpu.VMEM((_RP,), jnp.int32),
        pltpu.VMEM((_RP,), jnp.int32),
        pltpu.VMEM((_RP,), jnp.int32),
        pltpu.VMEM((_RP,), jnp.int32),
        pltpu.VMEM((_RP,), jnp.int32),
        pltpu.VMEM((_RP, _D), jnp.float32),
        pltpu.VMEM((_RP, _D), jnp.float32),
        pltpu.VMEM((_RP, _D), jnp.float32),
        pltpu.VMEM((_L * _C, _PASS_B), jnp.float32),
        pltpu.SemaphoreType.DMA,
    ],
)
def _gather_concat(iid_hbm, a0_hbm, a1_hbm, lrow_hbm, bcol_hbm,
                   item_t, attr_t, out_hbm,
                   ii_v, i0_v, i1_v, lr_v, bc_v, r0_v, r1_v, r2_v, out_v, sem):
    wid = lax.axis_index("s") * 2 + lax.axis_index("c")

    def pass_body(p, _):
        b0 = wid * _B_W + p * _PASS_B
        i0 = b0 * _L
        pltpu.sync_copy(iid_hbm.at[pl.ds(i0, _RP)], ii_v)
        pltpu.sync_copy(a0_hbm.at[pl.ds(i0, _RP)], i0_v)
        pltpu.sync_copy(a1_hbm.at[pl.ds(i0, _RP)], i1_v)
        pltpu.sync_copy(lrow_hbm.at[pl.ds(i0, _RP)], lr_v)
        pltpu.sync_copy(bcol_hbm.at[pl.ds(i0, _RP)], bc_v)
        c1 = pltpu.async_copy(item_t.at[ii_v], r0_v, sem)
        c2 = pltpu.async_copy(attr_t.at[i0_v], r1_v, sem)
        c3 = pltpu.async_copy(attr_t.at[i1_v], r2_v, sem)
        c1.wait()
        c2.wait()
        c3.wait()

        def group_body(g, _):
            j16 = lax.iota(jnp.int32, 16) + g * 16
            lr16 = lr_v[pl.ds(g * 16, 16)]
            bc16 = bc_v[pl.ds(g * 16, 16)] - b0
            for rows_v, c0 in ((r0_v, 0), (r1_v, _D), (r2_v, 2 * _D)):
                for d in range(_D):
                    v = plsc.load_gather(rows_v, [j16, jnp.full((16,), d, jnp.int32)])
                    plsc.store_scatter(out_v, [lr16 + (c0 + d), bc16], v)
            return 0

        lax.fori_loop(0, _GROUPS, group_body, 0)
        pltpu.sync_copy(out_v, out_hbm.at[:, pl.ds(b0, _PASS_B)])
        return 0

    lax.fori_loop(0, _NPASS, pass_body, 0)


def kernel(hist_iid_seq, hist_aid_seq, hist_rate_seq, hist_seq_len,
           iid_list, aid_list, lb_list,
           item_table, attr_table, rating_table):
    iid = iid_list.reshape(_BL).astype(jnp.int32)
    a0 = aid_list[:, :, 0].reshape(_BL).astype(jnp.int32)
    a1 = aid_list[:, :, 1].reshape(_BL).astype(jnp.int32)
    ar = jnp.arange(_BL, dtype=jnp.int32)
    lrow = (ar % _L) * _C
    bcol = ar // _L
    out4 = _gather_concat(iid, a0, a1, lrow, bcol, item_table, attr_table)
    out = out4.reshape(_L, _C, _B).transpose(2, 0, 1)
    return out, lb_list.reshape(_B, _L)
